# speedup vs baseline: 2.7792x; 2.7792x over previous
import jax
import jax.numpy as jnp
from jax import lax
from jax.experimental import pallas as pl
from jax.experimental.pallas import tpu as pltpu
from jax.experimental.pallas import tpu_sc as plsc

NC = 2
NS = 16
NW = NC * NS

B, H, S, D = 2, 16, 8192, 128
N = 4096
CHUNK = 256
NCHUNK = N // CHUNK


def _gather_body(x_hbm, idx_hbm, out_hbm, idx_v, offs_v, buf_v, gsem):
    wid = lax.axis_index("s") * NC + lax.axis_index("c")
    base_row = wid * S

    pltpu.sync_copy(idx_hbm, idx_v)

    def add_off(i, carry):
        sl = pl.ds(i * 16, 16)
        offs_v[sl] = idx_v[sl] + base_row
        return carry

    lax.fori_loop(0, N // 16, add_off, 0)

    out_base = wid * N
    for c in range(NCHUNK):
        offs_chunk = offs_v.at[pl.ds(c * CHUNK, CHUNK)]
        pltpu.async_copy(x_hbm.at[offs_chunk], buf_v, gsem).wait()
        pltpu.sync_copy(buf_v, out_hbm.at[pl.ds(out_base + c * CHUNK, CHUNK)])


@jax.jit
def _direct_slice(x_flat, idx):
    mesh = plsc.VectorSubcoreMesh(core_axis_name="c", subcore_axis_name="s")
    kern = pl.kernel(
        _gather_body,
        out_type=jax.ShapeDtypeStruct((B * H * N, D), jnp.float32),
        mesh=mesh,
        scratch_types=[
            pltpu.VMEM((N,), jnp.int32),
            pltpu.VMEM((N,), jnp.int32),
            pltpu.VMEM((CHUNK, D), jnp.float32),
            pltpu.SemaphoreType.DMA,
        ],
    )
    return kern(x_flat, idx)


def kernel(x, indices_to_select):
    idx = indices_to_select.astype(jnp.int32)
    x_flat = x.reshape(B * H * S, D)
    out_flat = _direct_slice(x_flat, idx)
    return out_flat.reshape(B, H, N, D)


# speedup vs baseline: 3.2697x; 1.1765x over previous
import jax
import jax.numpy as jnp
from jax import lax
from jax.experimental import pallas as pl
from jax.experimental.pallas import tpu as pltpu
from jax.experimental.pallas import tpu_sc as plsc

NC = 2
NS = 16
NW = NC * NS

B, H, S, D = 2, 16, 8192, 128
N = 4096
NBUF = 4
CHUNK = 128
NCHUNK = N // CHUNK
VPC = CHUNK // 16


def _gather_body(x_hbm, idx_hbm, out_hbm, idx_v, offs_v, bufs, gsems, ssems):
    wid = lax.axis_index("s") * NC + lax.axis_index("c")
    base_row = wid * S
    out_base = wid * N

    pltpu.sync_copy(idx_hbm, idx_v)

    def add_chunk(c):
        for i in range(VPC):
            sl = pl.ds(c * CHUNK + i * 16, 16)
            offs_v[sl] = idx_v[sl] + base_row

    def start_gather(c, b):
        pltpu.async_copy(
            x_hbm.at[offs_v.at[pl.ds(c * CHUNK, CHUNK)]], bufs[b], gsems[b])

    def wait_gather(b):
        pltpu.make_async_copy(
            x_hbm.at[offs_v.at[pl.ds(0, CHUNK)]], bufs[b], gsems[b]).wait()

    def start_scatter(c, b):
        pltpu.async_copy(
            bufs[b], out_hbm.at[pl.ds(out_base + c * CHUNK, CHUNK)], ssems[b])

    def wait_scatter(b):
        pltpu.make_async_copy(
            bufs[b], out_hbm.at[pl.ds(out_base, CHUNK)], ssems[b]).wait()

    def step(c, b, do_scatter_wait, gather_ahead):
        wait_gather(b)
        start_scatter(c, b)
        bprev = (b - 1) % NBUF
        if do_scatter_wait:
            wait_scatter(bprev)
        if gather_ahead:
            add_chunk(c + NBUF - 1)
            start_gather(c + NBUF - 1, bprev)

    for c in range(NBUF - 1):
        add_chunk(c)
        start_gather(c, c)

    step(0, 0, False, True)
    for c in range(1, NBUF):
        step(c, c % NBUF, True, True)

    def outer(i, carry):
        c4 = i * NBUF
        for b in range(NBUF):
            step(c4 + b, b, True, True)
        return carry

    lax.fori_loop(1, NCHUNK // NBUF - 1, outer, 0)

    last = NCHUNK - NBUF
    for c in range(last, NCHUNK):
        step(c, c % NBUF, True, gather_ahead=(c + NBUF - 1 < NCHUNK))
    wait_scatter((NCHUNK - 1) % NBUF)


@jax.jit
def _direct_slice(x_flat, idx):
    mesh = plsc.VectorSubcoreMesh(core_axis_name="c", subcore_axis_name="s")
    kern = pl.kernel(
        _gather_body,
        out_type=jax.ShapeDtypeStruct((B * H * N, D), jnp.float32),
        mesh=mesh,
        scratch_types=[
            pltpu.VMEM((N,), jnp.int32),
            pltpu.VMEM((N,), jnp.int32),
            [pltpu.VMEM((CHUNK, D), jnp.float32) for _ in range(NBUF)],
            [pltpu.SemaphoreType.DMA for _ in range(NBUF)],
            [pltpu.SemaphoreType.DMA for _ in range(NBUF)],
        ],
    )
    return kern(x_flat, idx)


def kernel(x, indices_to_select):
    idx = indices_to_select.astype(jnp.int32)
    x_flat = x.reshape(B * H * S, D)
    out_flat = _direct_slice(x_flat, idx)
    return out_flat.reshape(B, H, N, D)


# speedup vs baseline: 3.2752x; 1.0017x over previous
import jax
import jax.numpy as jnp
from jax import lax
from jax.experimental import pallas as pl
from jax.experimental.pallas import tpu as pltpu
from jax.experimental.pallas import tpu_sc as plsc

NC = 2
NS = 16
NW = NC * NS

B, H, S, D = 2, 16, 8192, 128
N = 4096
NBUF = 8
CHUNK = 64
NCHUNK = N // CHUNK
VPC = CHUNK // 16


def _gather_body(x_hbm, idx_hbm, out_hbm, idx_v, offs_v, bufs, gsems, ssems):
    wid = lax.axis_index("s") * NC + lax.axis_index("c")
    base_row = wid * S
    out_base = wid * N

    pltpu.sync_copy(idx_hbm, idx_v)

    def add_chunk(c):
        for i in range(VPC):
            sl = pl.ds(c * CHUNK + i * 16, 16)
            offs_v[sl] = idx_v[sl] + base_row

    def start_gather(c, b):
        pltpu.async_copy(
            x_hbm.at[offs_v.at[pl.ds(c * CHUNK, CHUNK)]], bufs[b], gsems[b])

    def wait_gather(b):
        pltpu.make_async_copy(
            x_hbm.at[offs_v.at[pl.ds(0, CHUNK)]], bufs[b], gsems[b]).wait()

    def start_scatter(c, b):
        pltpu.async_copy(
            bufs[b], out_hbm.at[pl.ds(out_base + c * CHUNK, CHUNK)], ssems[b])

    def wait_scatter(b):
        pltpu.make_async_copy(
            bufs[b], out_hbm.at[pl.ds(out_base, CHUNK)], ssems[b]).wait()

    def step(c, b, do_scatter_wait, gather_ahead):
        wait_gather(b)
        start_scatter(c, b)
        bprev = (b - 1) % NBUF
        if do_scatter_wait:
            wait_scatter(bprev)
        if gather_ahead:
            add_chunk(c + NBUF - 1)
            start_gather(c + NBUF - 1, bprev)

    for c in range(NBUF - 1):
        add_chunk(c)
        start_gather(c, c)

    step(0, 0, False, True)
    for c in range(1, NBUF):
        step(c, c % NBUF, True, True)

    def outer(i, carry):
        c4 = i * NBUF
        for b in range(NBUF):
            step(c4 + b, b, True, True)
        return carry

    lax.fori_loop(1, NCHUNK // NBUF - 1, outer, 0)

    last = NCHUNK - NBUF
    for c in range(last, NCHUNK):
        step(c, c % NBUF, True, gather_ahead=(c + NBUF - 1 < NCHUNK))
    wait_scatter((NCHUNK - 1) % NBUF)


@jax.jit
def _direct_slice(x_flat, idx):
    mesh = plsc.VectorSubcoreMesh(core_axis_name="c", subcore_axis_name="s")
    kern = pl.kernel(
        _gather_body,
        out_type=jax.ShapeDtypeStruct((B * H * N, D), jnp.float32),
        mesh=mesh,
        scratch_types=[
            pltpu.VMEM((N,), jnp.int32),
            pltpu.VMEM((N,), jnp.int32),
            [pltpu.VMEM((CHUNK, D), jnp.float32) for _ in range(NBUF)],
            [pltpu.SemaphoreType.DMA for _ in range(NBUF)],
            [pltpu.SemaphoreType.DMA for _ in range(NBUF)],
        ],
    )
    return kern(x_flat, idx)


def kernel(x, indices_to_select):
    idx = indices_to_select.astype(jnp.int32)
    x_flat = x.reshape(B * H * S, D)
    out_flat = _direct_slice(x_flat, idx)
    return out_flat.reshape(B, H, N, D)
